# baseline (speedup 1.0000x reference)
import functools

import jax
import jax.numpy as jnp
from jax import lax
from jax.experimental import pallas as pl
from jax.experimental.pallas import tpu as pltpu
from jax.experimental.pallas import tpu_sc as plsc

EPS = 1e-6
BN_EPS = 1e-5
SLOPE = 0.2
KNN = 16
NB = 128
MB = NB * KNN
KNB = 1024


def _knn_body(x2_ref, idx_ref, *, n, nb, k):
    b = pl.program_id(0)
    q = pl.program_id(1)
    P = x2_ref[0]
    R = jnp.transpose(x2_ref[0, :, pl.ds(q * nb, nb)], (1, 0))
    r = [R[:, d:d + 1] for d in range(3)]
    p = [P[d:d + 1, :] for d in range(3)]
    inner = jnp.dot(R, P, preferred_element_type=jnp.float32)
    xxr = r[0] * r[0] + r[1] * r[1] + r[2] * r[2]
    xxc = p[0] * p[0] + p[1] * p[1] + p[2] * p[2]
    D = 2.0 * inner - xxr - xxc
    iota = lax.broadcasted_iota(jnp.int32, (nb, n), 1)
    cols = []
    for _ in range(k):
        m = jnp.max(D, axis=1, keepdims=True)
        cand = jnp.where(D == m, iota, n)
        am = jnp.min(cand, axis=1, keepdims=True)
        cols.append(am)
        D = jnp.where(iota == am, -jnp.inf, D)
    idx_ref[...] = jnp.concatenate(cols, axis=1) + b * n


def _build_sc_gather(bn_total, total, k):
    info = plsc.get_sparse_core_info()
    NC, NSC, L = info.num_cores, info.num_subcores, info.num_lanes
    NW = NC * NSC
    per = total // NW
    nv = per // L
    blocks_per_tile = per // MB
    mesh = plsc.VectorSubcoreMesh(core_axis_name="c", subcore_axis_name="s")

    @functools.partial(
        pl.kernel,
        mesh=mesh,
        compiler_params=pltpu.CompilerParams(needs_layout_passes=False),
        out_type=jax.ShapeDtypeStruct((total * 6,), jnp.float32),
        scratch_types=[
            pltpu.VMEM((bn_total,), jnp.float32),
            pltpu.VMEM((bn_total,), jnp.float32),
            pltpu.VMEM((bn_total,), jnp.float32),
            pltpu.VMEM((per,), jnp.int32),
            pltpu.VMEM((per * 6,), jnp.float32),
        ],
    )
    def sc_gather(cx_hbm, cy_hbm, cz_hbm, idx_hbm, out_hbm, cx, cy, cz, idxv, obuf):
        wid = lax.axis_index("s") * NC + lax.axis_index("c")
        pltpu.sync_copy(cx_hbm, cx)
        pltpu.sync_copy(cy_hbm, cy)
        pltpu.sync_copy(cz_hbm, cz)
        base = wid * per
        pltpu.sync_copy(idx_hbm.at[pl.ds(base, per)], idxv)
        lane = lax.iota(jnp.int32, L)

        def body(t, carry):
            vi = idxv[pl.ds(t * L, L)]
            gx = plsc.load_gather(cx, [vi])
            gy = plsc.load_gather(cy, [vi])
            gz = plsc.load_gather(cz, [vi])
            rows = base + t * L + lane
            ci = rows // k
            hx = plsc.load_gather(cx, [ci])
            hy = plsc.load_gather(cy, [ci])
            hz = plsc.load_gather(cz, [ci])
            pv = (t >> 7) * (6 * MB) + lane * NB + (t & 127)
            plsc.store_scatter(obuf, [pv], gx - hx)
            plsc.store_scatter(obuf, [pv + MB], gy - hy)
            plsc.store_scatter(obuf, [pv + 2 * MB], gz - hz)
            plsc.store_scatter(obuf, [pv + 3 * MB], hx)
            plsc.store_scatter(obuf, [pv + 4 * MB], hy)
            plsc.store_scatter(obuf, [pv + 5 * MB], hz)
            return carry

        lax.fori_loop(0, nv, body, 0)
        pltpu.sync_copy(obuf, out_hbm.at[pl.ds(base * 6, per * 6)])

    return sc_gather


def _bf(v):
    return v.astype(jnp.bfloat16).astype(jnp.float32)


def _layer1_pd(feat, w1):
    w0 = _bf(w1[:, 0:1])
    w1c = _bf(w1[:, 1:2])
    out = []
    for d in range(3):
        e = _bf(feat[d:d + 1, :])
        c = _bf(feat[3 + d:4 + d, :])
        out.append(e * w0 + c * w1c)
    return out


def _norm3(v):
    return jnp.sqrt(v[0] * v[0] + v[1] * v[1] + v[2] * v[2])


def _bn_leaky(p, dvec, n, s, minv):
    mean = s[:, 0:1] * minv
    var = s[:, 1:2] * minv - mean * mean
    f = (n - mean) * lax.rsqrt(var + BN_EPS) / n
    rawdot = p[0] * dvec[0] + p[1] * dvec[1] + p[2] * dvec[2]
    dotp = f * rawdot
    dsq = dvec[0] * dvec[0] + dvec[1] * dvec[1] + dvec[2] * dvec[2]
    coef = (1.0 - SLOPE) * jnp.where(dotp >= 0, 0.0, 1.0) * (dotp / (dsq + EPS))
    return [f * p[d] - coef * dvec[d] for d in range(3)]


def _apply_l1(feat, w1f, w1d, s1, minv):
    p1 = _layer1_pd(feat, w1f)
    d1 = _layer1_pd(feat, w1d)
    n1 = _norm3(p1) + EPS
    return _bn_leaky(p1, d1, n1, s1, minv)


def _mm(w, x):
    return jnp.dot(w, x, preferred_element_type=jnp.float32)


def _accumulate(s_ref, n):
    part = jnp.concatenate(
        [jnp.sum(n, axis=1, keepdims=True),
         jnp.sum(n * n, axis=1, keepdims=True)], axis=1)

    @pl.when(pl.program_id(0) == 0)
    def _():
        s_ref[...] = jnp.zeros_like(s_ref)

    s_ref[...] += part


def _stats1_body(feat_ref, w1f_ref, s_ref):
    p1 = _layer1_pd(feat_ref[0], w1f_ref[...])
    n1 = _norm3(p1) + EPS
    _accumulate(s_ref, n1)


def _stats2_body(feat_ref, w1f_ref, w1d_ref, w2f_ref, s1_ref, s_ref, *, minv):
    x1 = _apply_l1(feat_ref[0], w1f_ref[...], w1d_ref[...], s1_ref[...], minv)
    p2 = [_mm(w2f_ref[...], x1[d]) for d in range(3)]
    n2 = _norm3(p2) + EPS
    _accumulate(s_ref, n2)


def _final_body(feat_ref, w1f_ref, w1d_ref, w2f_ref, w2d_ref, s1_ref, s2_ref,
                out_ref, *, minv, k):
    x1 = _apply_l1(feat_ref[0], w1f_ref[...], w1d_ref[...], s1_ref[...], minv)
    p2 = [_mm(w2f_ref[...], x1[d]) for d in range(3)]
    d2 = [_mm(w2d_ref[...], x1[d]) for d in range(3)]
    n2 = _norm3(p2) + EPS
    x2 = _bn_leaky(p2, d2, n2, s2_ref[...], minv)
    pooled = []
    for d in range(3):
        acc = x2[d][:, 0:NB]
        for j in range(1, k):
            acc = acc + x2[d][:, j * NB:(j + 1) * NB]
        pooled.append(acc * (1.0 / k))
    out_ref[...] = jnp.stack(pooled, axis=1)[None]


def kernel(x, W1_feat, W1_dir, W2_feat, W2_dir):
    B, C, _, N = x.shape
    assert C == 1
    O = W1_feat.shape[0]
    k = KNN
    nblk = N // NB
    x2 = x.reshape(B, 3, N)

    knblk = N // KNB
    idx = pl.pallas_call(
        functools.partial(_knn_body, n=N, nb=KNB, k=k),
        grid=(B, knblk),
        in_specs=[
            pl.BlockSpec((1, 3, N), lambda b, q: (b, 0, 0)),
        ],
        out_specs=pl.BlockSpec(
            (KNB, k), lambda b, q, _kb=knblk: (b * _kb + q, 0)),
        out_shape=jax.ShapeDtypeStruct((B * N, k), jnp.int32),
    )(x2)

    coordsT = jnp.transpose(x2, (1, 0, 2)).reshape(3, B * N)
    total = B * N * k
    gsteps = total // MB
    feat = _build_sc_gather(B * N, total, k)(
        coordsT[0], coordsT[1], coordsT[2], idx.reshape(-1))
    feat = feat.reshape(gsteps, 6, MB)

    minv = 1.0 / float(total)

    feat_spec = pl.BlockSpec((1, 6, MB), lambda g: (g, 0, 0))
    w1_spec = pl.BlockSpec((O, 2), lambda g: (0, 0))
    w2_spec = pl.BlockSpec((O, O), lambda g: (0, 0))
    s_spec = pl.BlockSpec((O, 2), lambda g: (0, 0))
    s_shape = jax.ShapeDtypeStruct((O, 2), jnp.float32)

    stats1 = pl.pallas_call(
        _stats1_body,
        grid=(gsteps,),
        in_specs=[feat_spec, w1_spec],
        out_specs=s_spec,
        out_shape=s_shape,
    )(feat, W1_feat)

    stats2 = pl.pallas_call(
        functools.partial(_stats2_body, minv=minv),
        grid=(gsteps,),
        in_specs=[feat_spec, w1_spec, w1_spec, w2_spec, s_spec],
        out_specs=s_spec,
        out_shape=s_shape,
    )(feat, W1_feat, W1_dir, W2_feat, stats1)

    out = pl.pallas_call(
        functools.partial(_final_body, minv=minv, k=k),
        grid=(gsteps,),
        in_specs=[feat_spec, w1_spec, w1_spec, w2_spec, w2_spec, s_spec, s_spec],
        out_specs=pl.BlockSpec(
            (1, O, 3, NB),
            lambda g, _nblk=nblk: (g // _nblk, 0, 0, g % _nblk)),
        out_shape=jax.ShapeDtypeStruct((B, O, 3, N), jnp.float32),
    )(feat, W1_feat, W1_dir, W2_feat, W2_dir, stats1, stats2)

    return out


# speedup vs baseline: 1.1721x; 1.1721x over previous
import functools

import jax
import jax.numpy as jnp
from jax import lax
from jax.experimental import pallas as pl
from jax.experimental.pallas import tpu as pltpu
from jax.experimental.pallas import tpu_sc as plsc

EPS = 1e-6
BN_EPS = 1e-5
SLOPE = 0.2
KNN = 16
NB = 256
MB = NB * KNN
KNB = 512


def _knn_body(x2_ref, idx_ref, *, n, nb, k):
    b = pl.program_id(0)
    q = pl.program_id(1)
    P = x2_ref[0]
    R = jnp.transpose(x2_ref[0, :, pl.ds(q * nb, nb)], (1, 0))
    r = [R[:, d:d + 1] for d in range(3)]
    p = [P[d:d + 1, :] for d in range(3)]
    inner = jnp.dot(R, P, preferred_element_type=jnp.float32)
    xxr = r[0] * r[0] + r[1] * r[1] + r[2] * r[2]
    xxc = p[0] * p[0] + p[1] * p[1] + p[2] * p[2]
    D = 2.0 * inner - xxr - xxc
    iota = lax.broadcasted_iota(jnp.int32, (nb, n), 1)
    cols = []
    for _ in range(k):
        m = jnp.max(D, axis=1, keepdims=True)
        cand = jnp.where(D == m, iota, n)
        am = jnp.min(cand, axis=1, keepdims=True)
        cols.append(am)
        D = jnp.where(iota == am, -jnp.inf, D)
    idx_ref[...] = jnp.concatenate(cols, axis=1) + b * n


def _build_sc_gather(bn_total, total, k):
    info = plsc.get_sparse_core_info()
    NC, NSC, L = info.num_cores, info.num_subcores, info.num_lanes
    NW = NC * NSC
    per = total // NW
    nv = per // L
    blocks_per_tile = per // MB
    mesh = plsc.VectorSubcoreMesh(core_axis_name="c", subcore_axis_name="s")

    @functools.partial(
        pl.kernel,
        mesh=mesh,
        compiler_params=pltpu.CompilerParams(needs_layout_passes=False),
        out_type=jax.ShapeDtypeStruct((total * 6,), jnp.float32),
        scratch_types=[
            pltpu.VMEM((bn_total,), jnp.float32),
            pltpu.VMEM((bn_total,), jnp.float32),
            pltpu.VMEM((bn_total,), jnp.float32),
            pltpu.VMEM((per,), jnp.int32),
            pltpu.VMEM((per * 6,), jnp.float32),
        ],
    )
    def sc_gather(cx_hbm, cy_hbm, cz_hbm, idx_hbm, out_hbm, cx, cy, cz, idxv, obuf):
        wid = lax.axis_index("s") * NC + lax.axis_index("c")
        pltpu.sync_copy(cx_hbm, cx)
        pltpu.sync_copy(cy_hbm, cy)
        pltpu.sync_copy(cz_hbm, cz)
        base = wid * per
        pltpu.sync_copy(idx_hbm.at[pl.ds(base, per)], idxv)
        lane = lax.iota(jnp.int32, L)

        def body(t, carry):
            vi = idxv[pl.ds(t * L, L)]
            gx = plsc.load_gather(cx, [vi])
            gy = plsc.load_gather(cy, [vi])
            gz = plsc.load_gather(cz, [vi])
            rows = base + t * L + lane
            ci = rows // k
            hx = plsc.load_gather(cx, [ci])
            hy = plsc.load_gather(cy, [ci])
            hz = plsc.load_gather(cz, [ci])
            tpb = MB // L
            pv = (t // tpb) * (6 * MB) + lane * NB + (t % tpb)
            plsc.store_scatter(obuf, [pv], gx - hx)
            plsc.store_scatter(obuf, [pv + MB], gy - hy)
            plsc.store_scatter(obuf, [pv + 2 * MB], gz - hz)
            plsc.store_scatter(obuf, [pv + 3 * MB], hx)
            plsc.store_scatter(obuf, [pv + 4 * MB], hy)
            plsc.store_scatter(obuf, [pv + 5 * MB], hz)
            return carry

        lax.fori_loop(0, nv, body, 0)
        pltpu.sync_copy(obuf, out_hbm.at[pl.ds(base * 6, per * 6)])

    return sc_gather


def _bf(v):
    return v.astype(jnp.bfloat16).astype(jnp.float32)


def _layer1_pd(feat, w1):
    w0 = _bf(w1[:, 0:1])
    w1c = _bf(w1[:, 1:2])
    out = []
    for d in range(3):
        e = _bf(feat[d:d + 1, :])
        c = _bf(feat[3 + d:4 + d, :])
        out.append(e * w0 + c * w1c)
    return out


def _norm3(v):
    return jnp.sqrt(v[0] * v[0] + v[1] * v[1] + v[2] * v[2])


def _bn_leaky(p, dvec, n, s, minv):
    mean = s[:, 0:1] * minv
    var = s[:, 1:2] * minv - mean * mean
    f = (n - mean) * lax.rsqrt(var + BN_EPS) / n
    rawdot = p[0] * dvec[0] + p[1] * dvec[1] + p[2] * dvec[2]
    dotp = f * rawdot
    dsq = dvec[0] * dvec[0] + dvec[1] * dvec[1] + dvec[2] * dvec[2]
    coef = (1.0 - SLOPE) * jnp.where(dotp >= 0, 0.0, 1.0) * (dotp / (dsq + EPS))
    return [f * p[d] - coef * dvec[d] for d in range(3)]


def _apply_l1(feat, w1f, w1d, s1, minv):
    p1 = _layer1_pd(feat, w1f)
    d1 = _layer1_pd(feat, w1d)
    n1 = _norm3(p1) + EPS
    return _bn_leaky(p1, d1, n1, s1, minv)


def _mm(w, x):
    return jnp.dot(w, x, preferred_element_type=jnp.float32)


def _accumulate(s_ref, n):
    part = jnp.concatenate(
        [jnp.sum(n, axis=1, keepdims=True),
         jnp.sum(n * n, axis=1, keepdims=True)], axis=1)

    @pl.when(pl.program_id(0) == 0)
    def _():
        s_ref[...] = jnp.zeros_like(s_ref)

    s_ref[...] += part


def _stats1_body(feat_ref, w1f_ref, s_ref):
    p1 = _layer1_pd(feat_ref[0], w1f_ref[...])
    n1 = _norm3(p1) + EPS
    _accumulate(s_ref, n1)


def _stats2_body(feat_ref, w1f_ref, w1d_ref, w2f_ref, s1_ref, s_ref, *, minv):
    x1 = _apply_l1(feat_ref[0], w1f_ref[...], w1d_ref[...], s1_ref[...], minv)
    p2 = [_mm(w2f_ref[...], x1[d]) for d in range(3)]
    n2 = _norm3(p2) + EPS
    _accumulate(s_ref, n2)


def _final_body(feat_ref, w1f_ref, w1d_ref, w2f_ref, w2d_ref, s1_ref, s2_ref,
                out_ref, *, minv, k):
    x1 = _apply_l1(feat_ref[0], w1f_ref[...], w1d_ref[...], s1_ref[...], minv)
    p2 = [_mm(w2f_ref[...], x1[d]) for d in range(3)]
    d2 = [_mm(w2d_ref[...], x1[d]) for d in range(3)]
    n2 = _norm3(p2) + EPS
    x2 = _bn_leaky(p2, d2, n2, s2_ref[...], minv)
    pooled = []
    for d in range(3):
        acc = x2[d][:, 0:NB]
        for j in range(1, k):
            acc = acc + x2[d][:, j * NB:(j + 1) * NB]
        pooled.append(acc * (1.0 / k))
    out_ref[...] = jnp.stack(pooled, axis=1)[None]


def kernel(x, W1_feat, W1_dir, W2_feat, W2_dir):
    B, C, _, N = x.shape
    assert C == 1
    O = W1_feat.shape[0]
    k = KNN
    nblk = N // NB
    x2 = x.reshape(B, 3, N)

    knblk = N // KNB
    idx = pl.pallas_call(
        functools.partial(_knn_body, n=N, nb=KNB, k=k),
        grid=(B, knblk),
        in_specs=[
            pl.BlockSpec((1, 3, N), lambda b, q: (b, 0, 0)),
        ],
        out_specs=pl.BlockSpec(
            (KNB, k), lambda b, q, _kb=knblk: (b * _kb + q, 0)),
        out_shape=jax.ShapeDtypeStruct((B * N, k), jnp.int32),
    )(x2)

    coordsT = jnp.transpose(x2, (1, 0, 2)).reshape(3, B * N)
    total = B * N * k
    gsteps = total // MB
    feat = _build_sc_gather(B * N, total, k)(
        coordsT[0], coordsT[1], coordsT[2], idx.reshape(-1))
    feat = feat.reshape(gsteps, 6, MB)

    minv = 1.0 / float(total)

    feat_spec = pl.BlockSpec((1, 6, MB), lambda g: (g, 0, 0))
    w1_spec = pl.BlockSpec((O, 2), lambda g: (0, 0))
    w2_spec = pl.BlockSpec((O, O), lambda g: (0, 0))
    s_spec = pl.BlockSpec((O, 2), lambda g: (0, 0))
    s_shape = jax.ShapeDtypeStruct((O, 2), jnp.float32)

    stats1 = pl.pallas_call(
        _stats1_body,
        grid=(gsteps,),
        in_specs=[feat_spec, w1_spec],
        out_specs=s_spec,
        out_shape=s_shape,
    )(feat, W1_feat)

    stats2 = pl.pallas_call(
        functools.partial(_stats2_body, minv=minv),
        grid=(gsteps,),
        in_specs=[feat_spec, w1_spec, w1_spec, w2_spec, s_spec],
        out_specs=s_spec,
        out_shape=s_shape,
    )(feat, W1_feat, W1_dir, W2_feat, stats1)

    out = pl.pallas_call(
        functools.partial(_final_body, minv=minv, k=k),
        grid=(gsteps,),
        in_specs=[feat_spec, w1_spec, w1_spec, w2_spec, w2_spec, s_spec, s_spec],
        out_specs=pl.BlockSpec(
            (1, O, 3, NB),
            lambda g, _nblk=nblk: (g // _nblk, 0, 0, g % _nblk)),
        out_shape=jax.ShapeDtypeStruct((B, O, 3, N), jnp.float32),
    )(feat, W1_feat, W1_dir, W2_feat, W2_dir, stats1, stats2)

    return out


# speedup vs baseline: 1.1819x; 1.0083x over previous
import functools

import jax
import jax.numpy as jnp
from jax import lax
from jax.experimental import pallas as pl
from jax.experimental.pallas import tpu as pltpu
from jax.experimental.pallas import tpu_sc as plsc

EPS = 1e-6
BN_EPS = 1e-5
SLOPE = 0.2
KNN = 16
NB = 512
MB = NB * KNN
KNB = 512


def _knn_body(x2_ref, idx_ref, *, n, nb, k):
    b = pl.program_id(0)
    q = pl.program_id(1)
    P = x2_ref[0]
    R = jnp.transpose(x2_ref[0, :, pl.ds(q * nb, nb)], (1, 0))
    r = [R[:, d:d + 1] for d in range(3)]
    p = [P[d:d + 1, :] for d in range(3)]
    inner = jnp.dot(R, P, preferred_element_type=jnp.float32)
    xxr = r[0] * r[0] + r[1] * r[1] + r[2] * r[2]
    xxc = p[0] * p[0] + p[1] * p[1] + p[2] * p[2]
    D = 2.0 * inner - xxr - xxc
    iota = lax.broadcasted_iota(jnp.int32, (nb, n), 1)
    cols = []
    for _ in range(k):
        m = jnp.max(D, axis=1, keepdims=True)
        cand = jnp.where(D == m, iota, n)
        am = jnp.min(cand, axis=1, keepdims=True)
        cols.append(am)
        D = jnp.where(iota == am, -jnp.inf, D)
    idx_ref[...] = jnp.concatenate(cols, axis=1) + b * n


def _build_sc_gather(bn_total, total, k):
    info = plsc.get_sparse_core_info()
    NC, NSC, L = info.num_cores, info.num_subcores, info.num_lanes
    NW = NC * NSC
    per = total // NW
    nv = per // L
    blocks_per_tile = per // MB
    mesh = plsc.VectorSubcoreMesh(core_axis_name="c", subcore_axis_name="s")

    @functools.partial(
        pl.kernel,
        mesh=mesh,
        compiler_params=pltpu.CompilerParams(needs_layout_passes=False),
        out_type=jax.ShapeDtypeStruct((total * 6,), jnp.float32),
        scratch_types=[
            pltpu.VMEM((bn_total,), jnp.float32),
            pltpu.VMEM((bn_total,), jnp.float32),
            pltpu.VMEM((bn_total,), jnp.float32),
            pltpu.VMEM((per,), jnp.int32),
            pltpu.VMEM((per * 6,), jnp.float32),
        ],
    )
    def sc_gather(cx_hbm, cy_hbm, cz_hbm, idx_hbm, out_hbm, cx, cy, cz, idxv, obuf):
        wid = lax.axis_index("s") * NC + lax.axis_index("c")
        pltpu.sync_copy(cx_hbm, cx)
        pltpu.sync_copy(cy_hbm, cy)
        pltpu.sync_copy(cz_hbm, cz)
        base = wid * per
        pltpu.sync_copy(idx_hbm.at[pl.ds(base, per)], idxv)
        lane = lax.iota(jnp.int32, L)

        def body(t, carry):
            vi = idxv[pl.ds(t * L, L)]
            gx = plsc.load_gather(cx, [vi])
            gy = plsc.load_gather(cy, [vi])
            gz = plsc.load_gather(cz, [vi])
            rows = base + t * L + lane
            ci = rows // k
            hx = plsc.load_gather(cx, [ci])
            hy = plsc.load_gather(cy, [ci])
            hz = plsc.load_gather(cz, [ci])
            tpb = MB // L
            pv = (t // tpb) * (6 * MB) + lane * NB + (t % tpb)
            plsc.store_scatter(obuf, [pv], gx - hx)
            plsc.store_scatter(obuf, [pv + MB], gy - hy)
            plsc.store_scatter(obuf, [pv + 2 * MB], gz - hz)
            plsc.store_scatter(obuf, [pv + 3 * MB], hx)
            plsc.store_scatter(obuf, [pv + 4 * MB], hy)
            plsc.store_scatter(obuf, [pv + 5 * MB], hz)
            return carry

        lax.fori_loop(0, nv, body, 0)
        pltpu.sync_copy(obuf, out_hbm.at[pl.ds(base * 6, per * 6)])

    return sc_gather


def _bf(v):
    return v.astype(jnp.bfloat16).astype(jnp.float32)


def _layer1_pd(feat, w1):
    w0 = _bf(w1[:, 0:1])
    w1c = _bf(w1[:, 1:2])
    out = []
    for d in range(3):
        e = _bf(feat[d:d + 1, :])
        c = _bf(feat[3 + d:4 + d, :])
        out.append(e * w0 + c * w1c)
    return out


def _norm3(v):
    return jnp.sqrt(v[0] * v[0] + v[1] * v[1] + v[2] * v[2])


def _bn_leaky(p, dvec, n, s, minv):
    mean = s[:, 0:1] * minv
    var = s[:, 1:2] * minv - mean * mean
    f = (n - mean) * lax.rsqrt(var + BN_EPS) / n
    rawdot = p[0] * dvec[0] + p[1] * dvec[1] + p[2] * dvec[2]
    dotp = f * rawdot
    dsq = dvec[0] * dvec[0] + dvec[1] * dvec[1] + dvec[2] * dvec[2]
    coef = (1.0 - SLOPE) * jnp.where(dotp >= 0, 0.0, 1.0) * (dotp / (dsq + EPS))
    return [f * p[d] - coef * dvec[d] for d in range(3)]


def _apply_l1(feat, w1f, w1d, s1, minv):
    p1 = _layer1_pd(feat, w1f)
    d1 = _layer1_pd(feat, w1d)
    n1 = _norm3(p1) + EPS
    return _bn_leaky(p1, d1, n1, s1, minv)


def _mm(w, x):
    return jnp.dot(w, x, preferred_element_type=jnp.float32)


def _accumulate(s_ref, n):
    part = jnp.concatenate(
        [jnp.sum(n, axis=1, keepdims=True),
         jnp.sum(n * n, axis=1, keepdims=True)], axis=1)

    @pl.when(pl.program_id(0) == 0)
    def _():
        s_ref[...] = jnp.zeros_like(s_ref)

    s_ref[...] += part


def _stats1_body(feat_ref, w1f_ref, s_ref):
    p1 = _layer1_pd(feat_ref[0], w1f_ref[...])
    n1 = _norm3(p1) + EPS
    _accumulate(s_ref, n1)


def _stats2_body(feat_ref, w1f_ref, w1d_ref, w2f_ref, s1_ref, s_ref, *, minv):
    x1 = _apply_l1(feat_ref[0], w1f_ref[...], w1d_ref[...], s1_ref[...], minv)
    p2 = [_mm(w2f_ref[...], x1[d]) for d in range(3)]
    n2 = _norm3(p2) + EPS
    _accumulate(s_ref, n2)


def _final_body(feat_ref, w1f_ref, w1d_ref, w2f_ref, w2d_ref, s1_ref, s2_ref,
                out_ref, *, minv, k):
    x1 = _apply_l1(feat_ref[0], w1f_ref[...], w1d_ref[...], s1_ref[...], minv)
    p2 = [_mm(w2f_ref[...], x1[d]) for d in range(3)]
    d2 = [_mm(w2d_ref[...], x1[d]) for d in range(3)]
    n2 = _norm3(p2) + EPS
    x2 = _bn_leaky(p2, d2, n2, s2_ref[...], minv)
    pooled = []
    for d in range(3):
        acc = x2[d][:, 0:NB]
        for j in range(1, k):
            acc = acc + x2[d][:, j * NB:(j + 1) * NB]
        pooled.append(acc * (1.0 / k))
    out_ref[...] = jnp.stack(pooled, axis=1)[None]


def kernel(x, W1_feat, W1_dir, W2_feat, W2_dir):
    B, C, _, N = x.shape
    assert C == 1
    O = W1_feat.shape[0]
    k = KNN
    nblk = N // NB
    x2 = x.reshape(B, 3, N)

    knblk = N // KNB
    idx = pl.pallas_call(
        functools.partial(_knn_body, n=N, nb=KNB, k=k),
        grid=(B, knblk),
        in_specs=[
            pl.BlockSpec((1, 3, N), lambda b, q: (b, 0, 0)),
        ],
        out_specs=pl.BlockSpec(
            (KNB, k), lambda b, q, _kb=knblk: (b * _kb + q, 0)),
        out_shape=jax.ShapeDtypeStruct((B * N, k), jnp.int32),
    )(x2)

    coordsT = jnp.transpose(x2, (1, 0, 2)).reshape(3, B * N)
    total = B * N * k
    gsteps = total // MB
    feat = _build_sc_gather(B * N, total, k)(
        coordsT[0], coordsT[1], coordsT[2], idx.reshape(-1))
    feat = feat.reshape(gsteps, 6, MB)

    minv = 1.0 / float(total)

    feat_spec = pl.BlockSpec((1, 6, MB), lambda g: (g, 0, 0))
    w1_spec = pl.BlockSpec((O, 2), lambda g: (0, 0))
    w2_spec = pl.BlockSpec((O, O), lambda g: (0, 0))
    s_spec = pl.BlockSpec((O, 2), lambda g: (0, 0))
    s_shape = jax.ShapeDtypeStruct((O, 2), jnp.float32)

    stats1 = pl.pallas_call(
        _stats1_body,
        grid=(gsteps,),
        in_specs=[feat_spec, w1_spec],
        out_specs=s_spec,
        out_shape=s_shape,
    )(feat, W1_feat)

    stats2 = pl.pallas_call(
        functools.partial(_stats2_body, minv=minv),
        grid=(gsteps,),
        in_specs=[feat_spec, w1_spec, w1_spec, w2_spec, s_spec],
        out_specs=s_spec,
        out_shape=s_shape,
    )(feat, W1_feat, W1_dir, W2_feat, stats1)

    out = pl.pallas_call(
        functools.partial(_final_body, minv=minv, k=k),
        grid=(gsteps,),
        in_specs=[feat_spec, w1_spec, w1_spec, w2_spec, w2_spec, s_spec, s_spec],
        out_specs=pl.BlockSpec(
            (1, O, 3, NB),
            lambda g, _nblk=nblk: (g // _nblk, 0, 0, g % _nblk)),
        out_shape=jax.ShapeDtypeStruct((B, O, 3, N), jnp.float32),
    )(feat, W1_feat, W1_dir, W2_feat, W2_dir, stats1, stats2)

    return out
